# baseline (speedup 1.0000x reference)
import functools

import jax
import jax.numpy as jnp
from jax import lax
from jax.experimental import pallas as pl
from jax.experimental.pallas import tpu as pltpu
from jax.experimental.pallas import tpu_sc as plsc

NC = 2
NS = 16
L = 16


def _dense_body(f_ref, wt_ref, wb_ref, p_ref, pb_ref, h_ref, a8_ref):
    h = jnp.dot(f_ref[...], wt_ref[...], preferred_element_type=jnp.float32)
    h = h + wb_ref[...]
    h_ref[...] = h
    a8_ref[...] = jnp.dot(h, p_ref[...], preferred_element_type=jnp.float32) + pb_ref[...]


def _edge_body(E, src_hbm, dst_hbm, a8f_hbm, exf_hbm, denomp_hbm,
               a8_v, srcv, dstv, ex4, idxb, zv, denom_sh):
    c = lax.axis_index("c")
    s = lax.axis_index("s")
    w = c * NS + s
    epw = E // (NC * NS)
    C = 80
    n_chunks = epw // C
    N4 = denom_sh.shape[0]

    pltpu.sync_copy(a8f_hbm, a8_v)

    zchunk = N4 // 10

    def _zfill(i, carry):
        zv[pl.ds(i * L, L)] = jnp.zeros((L,), jnp.float32)
        return carry
    lax.fori_loop(0, zchunk // L, _zfill, 0)

    @pl.when(s < 10)
    def _():
        pltpu.sync_copy(zv, denom_sh.at[pl.ds(s * zchunk, zchunk)])
    plsc.subcore_barrier()

    def _chunk(ch, carry):
        base = w * epw + ch * C
        pltpu.sync_copy(src_hbm.at[pl.ds(base, C)], srcv)
        pltpu.sync_copy(dst_hbm.at[pl.ds(base, C)], dstv)
        for g in range(C // L):
            s16 = srcv[pl.ds(g * L, L)]
            d16 = dstv[pl.ds(g * L, L)]
            for k in range(4):
                av = plsc.load_gather(a8_v, [s16 * 8 + k])
                bv = plsc.load_gather(a8_v, [d16 * 8 + (4 + k)])
                x = av + bv
                x = jnp.where(x >= 0.0, x, 0.01 * x)
                ex4[k, pl.ds(g * L, L)] = jnp.exp(x)
        for k in range(4):
            for g in range(C // L):
                idxb[pl.ds(g * L, L)] = srcv[pl.ds(g * L, L)] * 4 + k
            pltpu.sync_copy(ex4.at[k], exf_hbm.at[pl.ds(k * E + base, C)])
            pltpu.sync_copy(ex4.at[k], denom_sh.at[idxb], add=True)
        return carry
    lax.fori_loop(0, n_chunks, _chunk, 0)

    plsc.subcore_barrier()

    @pl.when(s < 10)
    def _():
        pltpu.sync_copy(denom_sh.at[pl.ds(s * zchunk, zchunk)], zv)
        pltpu.sync_copy(zv, denomp_hbm.at[pl.ds(c * N4 + s * zchunk, zchunk)])


def _agg_body(N, E, H, src_hbm, dst_hbm, exf_hbm, denomp_hbm, h_hbm, agg_hbm,
              denv, mv0, mv1, srcv, dstv, exv, alv, sidx0, sidx1, rows0, rows1,
              zb, acc_sh, sem_g0, sem_g1, sem_s0, sem_s1):
    c = lax.axis_index("c")
    s = lax.axis_index("s")
    N4 = N * 4
    C = 80
    SUPER = 800
    NSUB = SUPER // C
    ept = E // NS
    n_super = ept // SUPER
    lanes = lax.iota(jnp.int32, L)
    rows = (rows0, rows1)
    sidx = (sidx0, sidx1)
    sem_g = (sem_g0, sem_g1)
    sem_s = (sem_s0, sem_s1)

    SLAB = 400
    n_slab = N // SLAB

    for hp in range(2):
        k = 2 * c + hp

        def _slab(sl, carry):
            pltpu.sync_copy(denomp_hbm.at[pl.ds(sl * SLAB * 4, SLAB * 4)], mv0)
            pltpu.sync_copy(denomp_hbm.at[pl.ds(N4 + sl * SLAB * 4, SLAB * 4)], mv1)

            def _add(i, cc):
                mv0[pl.ds(i * L, L)] = mv0[pl.ds(i * L, L)] + mv1[pl.ds(i * L, L)]
                return cc
            lax.fori_loop(0, SLAB * 4 // L, _add, 0)

            def _compact(g, cc):
                idx = (g * L + lanes) * 4 + k
                denv[pl.ds(sl * SLAB + g * L, L)] = plsc.load_gather(mv0, [idx])
                return cc
            lax.fori_loop(0, SLAB // L, _compact, 0)
            return carry
        lax.fori_loop(0, n_slab, _slab, 0)

        ZR = zb.shape[0]
        n_zc = N // ZR

        def _zfill(i, cc):
            for j in range(H // L):
                zb[i, pl.ds(j * L, L)] = jnp.zeros((L,), jnp.float32)
            return cc
        lax.fori_loop(0, ZR, _zfill, 0)

        def _zero(z, cc):
            cid = s + NS * z

            @pl.when(cid < n_zc)
            def _():
                pltpu.sync_copy(zb, acc_sh.at[pl.ds(cid * ZR, ZR)])

            @pl.when(cid == n_zc)
            def _():
                pltpu.sync_copy(zb.at[pl.ds(0, N % ZR)],
                                acc_sh.at[pl.ds(n_zc * ZR, N % ZR)])
            return cc
        lax.fori_loop(0, n_zc // NS + 1, _zero, 0)
        plsc.subcore_barrier()

        def _super(si, carry):
            base = s * ept + si * SUPER
            pltpu.sync_copy(src_hbm.at[pl.ds(base, SUPER)], srcv)
            pltpu.sync_copy(dst_hbm.at[pl.ds(base, SUPER)], dstv)
            pltpu.sync_copy(exf_hbm.at[pl.ds(k * E + base, SUPER)], exv)

            def _alpha(g, cc):
                s16 = srcv[pl.ds(g * L, L)]
                dv = plsc.load_gather(denv, [s16])
                alv[pl.ds(g * L, L)] = exv[pl.ds(g * L, L)] / dv
                return cc
            lax.fori_loop(0, SUPER // L, _alpha, 0)

            pltpu.async_copy(h_hbm.at[dstv.at[pl.ds(0, C)]], rows[0], sem_g[0])

            def _pair(pr, cc):
                for b in (0, 1):
                    j = 2 * pr + b
                    o = 1 - b
                    pltpu.make_async_copy(
                        h_hbm.at[dstv.at[pl.ds(0, C)]], rows[b], sem_g[b]).wait()
                    if b == 0:
                        @pl.when(pr >= 1)
                        def _():
                            pltpu.make_async_copy(
                                rows[o], acc_sh.at[sidx[o]], sem_s[o]).wait()
                    else:
                        pltpu.make_async_copy(
                            rows[o], acc_sh.at[sidx[o]], sem_s[o]).wait()

                    @pl.when(j + 1 < NSUB)
                    def _():
                        pltpu.async_copy(
                            h_hbm.at[dstv.at[pl.ds((j + 1) * C, C)]],
                            rows[o], sem_g[o])

                    for g in range(C // L):
                        a16 = alv[pl.ds(j * C + g * L, L)]
                        for l in range(L):
                            bc = jnp.broadcast_to(a16[l], (L,))
                            i = g * L + l
                            for jj in range(H // L):
                                rows[b][i, pl.ds(jj * L, L)] = (
                                    rows[b][i, pl.ds(jj * L, L)] * bc)
                    for g in range(C // L):
                        sidx[b][pl.ds(g * L, L)] = srcv[pl.ds(j * C + g * L, L)]
                    pltpu.async_copy(rows[b], acc_sh.at[sidx[b]], sem_s[b],
                                     add=True)
                return cc
            lax.fori_loop(0, NSUB // 2, _pair, 0)
            pltpu.make_async_copy(rows[1], acc_sh.at[sidx[1]], sem_s[1]).wait()
            return carry
        lax.fori_loop(0, n_super, _super, 0)
        plsc.subcore_barrier()

        def _wout(z, cc):
            cid = s + NS * z

            @pl.when(cid < n_zc)
            def _():
                pltpu.sync_copy(acc_sh.at[pl.ds(cid * ZR, ZR)], zb)
                pltpu.sync_copy(zb, agg_hbm.at[k, pl.ds(cid * ZR, ZR)])

            @pl.when(cid == n_zc)
            def _():
                pltpu.sync_copy(acc_sh.at[pl.ds(n_zc * ZR, N % ZR)],
                                zb.at[pl.ds(0, N % ZR)])
                pltpu.sync_copy(zb.at[pl.ds(0, N % ZR)],
                                agg_hbm.at[k, pl.ds(n_zc * ZR, N % ZR)])
            return cc
        lax.fori_loop(0, n_zc // NS + 1, _wout, 0)
        plsc.subcore_barrier()


def _out_body(h_ref, agg_ref, o_ref):
    parts = [h_ref[...] + agg_ref[kk] for kk in range(agg_ref.shape[0])]
    o_ref[...] = jnp.concatenate(parts, axis=1)


def kernel(feats, adjacency_matrix, adj_ind, W, W_b, attn_w, attn_b):
    N, F = feats.shape
    H = W.shape[0]
    HEADS = attn_w.shape[0]
    E = adj_ind.shape[1]
    del adjacency_matrix

    src = adj_ind[0].astype(jnp.int32)
    dst = adj_ind[1].astype(jnp.int32)

    Wt = W.T
    wb2 = W_b.reshape(1, H)
    P = jnp.concatenate([attn_w[:, :H].T, attn_w[:, H:].T], axis=1)
    pb2 = jnp.concatenate([attn_b, jnp.zeros((HEADS,), jnp.float32)]).reshape(1, 2 * HEADS)

    R = 1000
    h, a8 = pl.pallas_call(
        _dense_body,
        grid=(N // R,),
        in_specs=[
            pl.BlockSpec((R, F), lambda i: (i, 0)),
            pl.BlockSpec((F, H), lambda i: (0, 0)),
            pl.BlockSpec((1, H), lambda i: (0, 0)),
            pl.BlockSpec((H, 2 * HEADS), lambda i: (0, 0)),
            pl.BlockSpec((1, 2 * HEADS), lambda i: (0, 0)),
        ],
        out_specs=[
            pl.BlockSpec((R, H), lambda i: (i, 0)),
            pl.BlockSpec((R, 2 * HEADS), lambda i: (i, 0)),
        ],
        out_shape=[
            jax.ShapeDtypeStruct((N, H), jnp.float32),
            jax.ShapeDtypeStruct((N, 2 * HEADS), jnp.float32),
        ],
    )(feats, Wt, wb2, P, pb2)

    a8f = a8.reshape(-1)

    mesh = plsc.VectorSubcoreMesh(core_axis_name="c", subcore_axis_name="s",
                                  num_cores=NC, num_subcores=NS)
    sc_params = pltpu.CompilerParams(needs_layout_passes=False)

    edge_k = pl.kernel(
        functools.partial(_edge_body, E),
        out_type=(
            jax.ShapeDtypeStruct((HEADS * E,), jnp.float32),
            jax.ShapeDtypeStruct((NC * N * HEADS,), jnp.float32),
        ),
        mesh=mesh,
        scratch_types=(
            pltpu.VMEM((N * 2 * HEADS,), jnp.float32),
            pltpu.VMEM((80,), jnp.int32),
            pltpu.VMEM((80,), jnp.int32),
            pltpu.VMEM((HEADS, 80), jnp.float32),
            pltpu.VMEM((80,), jnp.int32),
            pltpu.VMEM((N * HEADS // 10,), jnp.float32),
            pltpu.VMEM_SHARED((N * HEADS,), jnp.float32),
        ),
        compiler_params=sc_params,
    )
    exf, denomp = edge_k(src, dst, a8f)

    agg_k = pl.kernel(
        functools.partial(_agg_body, N, E, H),
        out_type=jax.ShapeDtypeStruct((HEADS, N, H), jnp.float32),
        mesh=mesh,
        scratch_types=(
            pltpu.VMEM((N,), jnp.float32),
            pltpu.VMEM((1600,), jnp.float32),
            pltpu.VMEM((1600,), jnp.float32),
            pltpu.VMEM((800,), jnp.int32),
            pltpu.VMEM((800,), jnp.int32),
            pltpu.VMEM((800,), jnp.float32),
            pltpu.VMEM((816,), jnp.float32),
            pltpu.VMEM((80,), jnp.int32),
            pltpu.VMEM((80,), jnp.int32),
            pltpu.VMEM((80, H), jnp.float32),
            pltpu.VMEM((80, H), jnp.float32),
            pltpu.VMEM((64, H), jnp.float32),
            pltpu.VMEM_SHARED((N, H), jnp.float32),
            pltpu.SemaphoreType.DMA,
            pltpu.SemaphoreType.DMA,
            pltpu.SemaphoreType.DMA,
            pltpu.SemaphoreType.DMA,
        ),
        compiler_params=sc_params,
    )
    agg = agg_k(src, dst, exf, denomp, h)

    out = pl.pallas_call(
        _out_body,
        grid=(N // R,),
        in_specs=[
            pl.BlockSpec((R, H), lambda i: (i, 0)),
            pl.BlockSpec((HEADS, R, H), lambda i: (0, i, 0)),
        ],
        out_specs=pl.BlockSpec((R, HEADS * H), lambda i: (i, 0)),
        out_shape=jax.ShapeDtypeStruct((N, HEADS * H), jnp.float32),
    )(h, agg)
    return out


# speedup vs baseline: 1.1885x; 1.1885x over previous
import functools

import jax
import jax.numpy as jnp
from jax import lax
from jax.experimental import pallas as pl
from jax.experimental.pallas import tpu as pltpu
from jax.experimental.pallas import tpu_sc as plsc

NC = 2
NS = 16
L = 16


def _dense_body(f_ref, wt_ref, wb_ref, p_ref, pb_ref, h_ref, a8_ref):
    h = jnp.dot(f_ref[...], wt_ref[...], preferred_element_type=jnp.float32)
    h = h + wb_ref[...]
    h_ref[...] = h
    a8_ref[...] = jnp.dot(h, p_ref[...], preferred_element_type=jnp.float32) + pb_ref[...]


def _edge_body(E, src_hbm, dst_hbm, a8f_hbm, exf_hbm, denomp_hbm,
               a8_v, srcv, dstv, ex4, idxb, zv, denom_sh):
    c = lax.axis_index("c")
    s = lax.axis_index("s")
    w = c * NS + s
    epw = E // (NC * NS)
    C = 80
    n_chunks = epw // C
    N4 = denom_sh.shape[0]

    pltpu.sync_copy(a8f_hbm, a8_v)

    zchunk = N4 // 10

    def _zfill(i, carry):
        zv[pl.ds(i * L, L)] = jnp.zeros((L,), jnp.float32)
        return carry
    lax.fori_loop(0, zchunk // L, _zfill, 0)

    @pl.when(s < 10)
    def _():
        pltpu.sync_copy(zv, denom_sh.at[pl.ds(s * zchunk, zchunk)])
    plsc.subcore_barrier()

    def _chunk(ch, carry):
        base = w * epw + ch * C
        pltpu.sync_copy(src_hbm.at[pl.ds(base, C)], srcv)
        pltpu.sync_copy(dst_hbm.at[pl.ds(base, C)], dstv)
        for g in range(C // L):
            s16 = srcv[pl.ds(g * L, L)]
            d16 = dstv[pl.ds(g * L, L)]
            for k in range(4):
                av = plsc.load_gather(a8_v, [s16 * 8 + k])
                bv = plsc.load_gather(a8_v, [d16 * 8 + (4 + k)])
                x = av + bv
                x = jnp.where(x >= 0.0, x, 0.01 * x)
                ex4[k, pl.ds(g * L, L)] = jnp.exp(x)
        for k in range(4):
            for g in range(C // L):
                idxb[pl.ds(g * L, L)] = srcv[pl.ds(g * L, L)] * 4 + k
            pltpu.sync_copy(ex4.at[k], exf_hbm.at[pl.ds(k * E + base, C)])
            pltpu.sync_copy(ex4.at[k], denom_sh.at[idxb], add=True)
        return carry
    lax.fori_loop(0, n_chunks, _chunk, 0)

    plsc.subcore_barrier()

    @pl.when(s < 10)
    def _():
        pltpu.sync_copy(denom_sh.at[pl.ds(s * zchunk, zchunk)], zv)
        pltpu.sync_copy(zv, denomp_hbm.at[pl.ds(c * N4 + s * zchunk, zchunk)])


def _agg_body(N, E, H, src_hbm, dst_hbm, exf_hbm, denomp_hbm, h_hbm, agg_hbm,
              srcv, dstv, exv, sidx0, sidx1, rows0, rows1, sbuf0, sbuf1,
              zb, dm0, dm1, acc_sh, sem_g0, sem_g1, sem_s0, sem_s1):
    c = lax.axis_index("c")
    s = lax.axis_index("s")
    N4 = N * 4
    C = 80
    SUPER = 800
    NSUB = SUPER // C
    ept = E // NS
    n_super = ept // SUPER
    lanes = lax.iota(jnp.int32, L)
    rows = (rows0, rows1)
    sbuf = (sbuf0, sbuf1)
    sidx = (sidx0, sidx1)
    sem_g = (sem_g0, sem_g1)
    sem_s = (sem_s0, sem_s1)

    ZR = zb.shape[0]
    n_zc = N // ZR
    TAIL = N % ZR

    for hp in range(2):
        k = 2 * c + hp

        def _zfill(i, cc):
            for j in range(H // L):
                zb[i, pl.ds(j * L, L)] = jnp.zeros((L,), jnp.float32)
            return cc
        lax.fori_loop(0, ZR, _zfill, 0)

        def _zero(z, cc):
            cid = s + NS * z

            @pl.when(cid < n_zc)
            def _():
                pltpu.sync_copy(zb, acc_sh.at[pl.ds(cid * ZR, ZR)])

            @pl.when(cid == n_zc)
            def _():
                pltpu.sync_copy(zb.at[pl.ds(0, TAIL)],
                                acc_sh.at[pl.ds(n_zc * ZR, TAIL)])
            return cc
        lax.fori_loop(0, n_zc // NS + 1, _zero, 0)
        plsc.subcore_barrier()

        def _super(si, carry):
            base = s * ept + si * SUPER
            pltpu.sync_copy(src_hbm.at[pl.ds(base, SUPER)], srcv)
            pltpu.sync_copy(dst_hbm.at[pl.ds(base, SUPER)], dstv)
            pltpu.sync_copy(exf_hbm.at[pl.ds(k * E + base, SUPER)],
                            exv.at[pl.ds(0, SUPER)])

            pltpu.async_copy(h_hbm.at[dstv.at[pl.ds(0, C)]], rows[0], sem_g[0])

            def _pair(pr, cc):
                for b in (0, 1):
                    j = 2 * pr + b
                    o = 1 - b
                    pltpu.make_async_copy(
                        h_hbm.at[dstv.at[pl.ds(0, C)]], rows[b], sem_g[b]).wait()

                    @pl.when(j + 1 < NSUB)
                    def _():
                        pltpu.async_copy(
                            h_hbm.at[dstv.at[pl.ds((j + 1) * C, C)]],
                            rows[o], sem_g[o])

                    @pl.when(pr >= 1)
                    def _():
                        pltpu.make_async_copy(
                            sbuf[b], acc_sh.at[sidx[b]], sem_s[b]).wait()

                    def _scale(i, c2):
                        ev = exv[pl.ds(j * C + i, L)]
                        bc = jnp.broadcast_to(ev[0], (L,))
                        for jj in range(H // L):
                            sbuf[b][i, pl.ds(jj * L, L)] = (
                                rows[b][i, pl.ds(jj * L, L)] * bc)
                        return c2
                    lax.fori_loop(0, C, _scale, 0)
                    for g in range(C // L):
                        sidx[b][pl.ds(g * L, L)] = srcv[pl.ds(j * C + g * L, L)]
                    pltpu.async_copy(sbuf[b], acc_sh.at[sidx[b]], sem_s[b],
                                     add=True)
                return cc
            lax.fori_loop(0, NSUB // 2, _pair, 0)
            pltpu.make_async_copy(sbuf[0], acc_sh.at[sidx[0]], sem_s[0]).wait()
            pltpu.make_async_copy(sbuf[1], acc_sh.at[sidx[1]], sem_s[1]).wait()
            return carry
        lax.fori_loop(0, n_super, _super, 0)
        plsc.subcore_barrier()

        def _wchunk(r0, nr, dlen):
            pltpu.sync_copy(acc_sh.at[pl.ds(r0, nr)], zb.at[pl.ds(0, nr)])
            pltpu.sync_copy(denomp_hbm.at[pl.ds(r0 * 4, dlen)],
                            dm0.at[pl.ds(0, dlen)])
            pltpu.sync_copy(denomp_hbm.at[pl.ds(N4 + r0 * 4, dlen)],
                            dm1.at[pl.ds(0, dlen)])
            for g in range(dlen // L):
                dm0[pl.ds(g * L, L)] = (dm0[pl.ds(g * L, L)]
                                        + dm1[pl.ds(g * L, L)])
            for g in range(nr // L):
                d16 = plsc.load_gather(dm0, [(g * L + lanes) * 4 + k])
                r16 = jnp.where(d16 > 0.0, 1.0 / d16, 0.0)
                for l in range(L):
                    bc = jnp.broadcast_to(r16[l], (L,))
                    for jj in range(H // L):
                        zb[g * L + l, pl.ds(jj * L, L)] = (
                            zb[g * L + l, pl.ds(jj * L, L)] * bc)
            pltpu.sync_copy(zb.at[pl.ds(0, nr)], agg_hbm.at[k, pl.ds(r0, nr)])

        def _wout(z, cc):
            cid = s + NS * z

            @pl.when(cid < n_zc)
            def _():
                _wchunk(cid * ZR, ZR, ZR * 4)

            @pl.when(cid == n_zc)
            def _():
                _wchunk(n_zc * ZR, TAIL, TAIL * 4)
            return cc
        lax.fori_loop(0, n_zc // NS + 1, _wout, 0)
        plsc.subcore_barrier()


def _out_body(h_ref, agg_ref, o_ref):
    parts = [h_ref[...] + agg_ref[kk] for kk in range(agg_ref.shape[0])]
    o_ref[...] = jnp.concatenate(parts, axis=1)


def kernel(feats, adjacency_matrix, adj_ind, W, W_b, attn_w, attn_b):
    N, F = feats.shape
    H = W.shape[0]
    HEADS = attn_w.shape[0]
    E = adj_ind.shape[1]
    del adjacency_matrix

    src = adj_ind[0].astype(jnp.int32)
    dst = adj_ind[1].astype(jnp.int32)

    Wt = W.T
    wb2 = W_b.reshape(1, H)
    P = jnp.concatenate([attn_w[:, :H].T, attn_w[:, H:].T], axis=1)
    pb2 = jnp.concatenate([attn_b, jnp.zeros((HEADS,), jnp.float32)]).reshape(1, 2 * HEADS)

    R = 1000
    h, a8 = pl.pallas_call(
        _dense_body,
        grid=(N // R,),
        in_specs=[
            pl.BlockSpec((R, F), lambda i: (i, 0)),
            pl.BlockSpec((F, H), lambda i: (0, 0)),
            pl.BlockSpec((1, H), lambda i: (0, 0)),
            pl.BlockSpec((H, 2 * HEADS), lambda i: (0, 0)),
            pl.BlockSpec((1, 2 * HEADS), lambda i: (0, 0)),
        ],
        out_specs=[
            pl.BlockSpec((R, H), lambda i: (i, 0)),
            pl.BlockSpec((R, 2 * HEADS), lambda i: (i, 0)),
        ],
        out_shape=[
            jax.ShapeDtypeStruct((N, H), jnp.float32),
            jax.ShapeDtypeStruct((N, 2 * HEADS), jnp.float32),
        ],
    )(feats, Wt, wb2, P, pb2)

    a8f = a8.reshape(-1)

    mesh = plsc.VectorSubcoreMesh(core_axis_name="c", subcore_axis_name="s",
                                  num_cores=NC, num_subcores=NS)
    sc_params = pltpu.CompilerParams(needs_layout_passes=False)

    edge_k = pl.kernel(
        functools.partial(_edge_body, E),
        out_type=(
            jax.ShapeDtypeStruct((HEADS * E,), jnp.float32),
            jax.ShapeDtypeStruct((NC * N * HEADS,), jnp.float32),
        ),
        mesh=mesh,
        scratch_types=(
            pltpu.VMEM((N * 2 * HEADS,), jnp.float32),
            pltpu.VMEM((80,), jnp.int32),
            pltpu.VMEM((80,), jnp.int32),
            pltpu.VMEM((HEADS, 80), jnp.float32),
            pltpu.VMEM((80,), jnp.int32),
            pltpu.VMEM((N * HEADS // 10,), jnp.float32),
            pltpu.VMEM_SHARED((N * HEADS,), jnp.float32),
        ),
        compiler_params=sc_params,
    )
    exf, denomp = edge_k(src, dst, a8f)

    agg_k = pl.kernel(
        functools.partial(_agg_body, N, E, H),
        out_type=jax.ShapeDtypeStruct((HEADS, N, H), jnp.float32),
        mesh=mesh,
        scratch_types=(
            pltpu.VMEM((800,), jnp.int32),
            pltpu.VMEM((800,), jnp.int32),
            pltpu.VMEM((816,), jnp.float32),
            pltpu.VMEM((80,), jnp.int32),
            pltpu.VMEM((80,), jnp.int32),
            pltpu.VMEM((80, H), jnp.float32),
            pltpu.VMEM((80, H), jnp.float32),
            pltpu.VMEM((80, H), jnp.float32),
            pltpu.VMEM((80, H), jnp.float32),
            pltpu.VMEM((32, H), jnp.float32),
            pltpu.VMEM((128,), jnp.float32),
            pltpu.VMEM((128,), jnp.float32),
            pltpu.VMEM_SHARED((N, H), jnp.float32),
            pltpu.SemaphoreType.DMA,
            pltpu.SemaphoreType.DMA,
            pltpu.SemaphoreType.DMA,
            pltpu.SemaphoreType.DMA,
        ),
        compiler_params=sc_params,
    )
    agg = agg_k(src, dst, exf, denomp, h)

    out = pl.pallas_call(
        _out_body,
        grid=(N // R,),
        in_specs=[
            pl.BlockSpec((R, H), lambda i: (i, 0)),
            pl.BlockSpec((HEADS, R, H), lambda i: (0, i, 0)),
        ],
        out_specs=pl.BlockSpec((R, HEADS * H), lambda i: (i, 0)),
        out_shape=jax.ShapeDtypeStruct((N, HEADS * H), jnp.float32),
    )(h, agg)
    return out


# speedup vs baseline: 1.2114x; 1.0193x over previous
import functools

import jax
import jax.numpy as jnp
from jax import lax
from jax.experimental import pallas as pl
from jax.experimental.pallas import tpu as pltpu
from jax.experimental.pallas import tpu_sc as plsc

NC = 2
NS = 16
L = 16


def _dense_body(f_ref, wt_ref, wb_ref, p_ref, pb_ref, h_ref, a8_ref):
    h = jnp.dot(f_ref[...], wt_ref[...], preferred_element_type=jnp.float32)
    h = h + wb_ref[...]
    h_ref[...] = h
    a8_ref[...] = jnp.dot(h, p_ref[...], preferred_element_type=jnp.float32) + pb_ref[...]


def _edge_body(E, src_hbm, dst_hbm, a8f_hbm, exf_hbm, denomp_hbm,
               a8_v, srcv, dstv, ex4, idxb, zv, denom_sh):
    c = lax.axis_index("c")
    s = lax.axis_index("s")
    w = c * NS + s
    epw = E // (NC * NS)
    C = 80
    n_chunks = epw // C
    N4 = denom_sh.shape[0]

    pltpu.sync_copy(a8f_hbm, a8_v)

    zchunk = N4 // 10

    def _zfill(i, carry):
        zv[pl.ds(i * L, L)] = jnp.zeros((L,), jnp.float32)
        return carry
    lax.fori_loop(0, zchunk // L, _zfill, 0)

    @pl.when(s < 10)
    def _():
        pltpu.sync_copy(zv, denom_sh.at[pl.ds(s * zchunk, zchunk)])
    plsc.subcore_barrier()

    def _chunk(ch, carry):
        base = w * epw + ch * C
        pltpu.sync_copy(src_hbm.at[pl.ds(base, C)], srcv)
        pltpu.sync_copy(dst_hbm.at[pl.ds(base, C)], dstv)
        for g in range(C // L):
            s16 = srcv[pl.ds(g * L, L)]
            d16 = dstv[pl.ds(g * L, L)]
            for k in range(4):
                av = plsc.load_gather(a8_v, [s16 * 8 + k])
                bv = plsc.load_gather(a8_v, [d16 * 8 + (4 + k)])
                x = av + bv
                x = jnp.where(x >= 0.0, x, 0.01 * x)
                ex4[k, pl.ds(g * L, L)] = jnp.exp(x)
        for k in range(4):
            for g in range(C // L):
                idxb[pl.ds(g * L, L)] = srcv[pl.ds(g * L, L)] * 4 + k
            pltpu.sync_copy(ex4.at[k], exf_hbm.at[pl.ds(k * E + base, C)])
            pltpu.sync_copy(ex4.at[k], denom_sh.at[idxb], add=True)
        return carry
    lax.fori_loop(0, n_chunks, _chunk, 0)

    plsc.subcore_barrier()

    @pl.when(s < 10)
    def _():
        pltpu.sync_copy(denom_sh.at[pl.ds(s * zchunk, zchunk)], zv)
        pltpu.sync_copy(zv, denomp_hbm.at[pl.ds(c * N4 + s * zchunk, zchunk)])


def _agg_body(N, E, H, src_hbm, dst_hbm, exf_hbm, denomp_hbm, h_hbm, agg_hbm,
              srcv, dstv, exv, sidx0, sidx1, rows0, rows1, sbuf0, sbuf1,
              zb, dm0, dm1, acc_sh, sem_g0, sem_g1, sem_s0, sem_s1):
    c = lax.axis_index("c")
    s = lax.axis_index("s")
    N4 = N * 4
    C = 80
    SUPER = 800
    NSUB = SUPER // C
    ept = E // NS
    n_super = ept // SUPER
    lanes = lax.iota(jnp.int32, L)
    rows = (rows0, rows1)
    sbuf = (sbuf0, sbuf1)
    sidx = (sidx0, sidx1)
    sem_g = (sem_g0, sem_g1)
    sem_s = (sem_s0, sem_s1)

    ZR = zb.shape[0]
    n_zc = N // ZR
    TAIL = N % ZR

    for hp in range(2):
        k = 2 * c + hp

        def _zfill(i, cc):
            for j in range(H // L):
                zb[i, pl.ds(j * L, L)] = jnp.zeros((L,), jnp.float32)
            return cc
        lax.fori_loop(0, ZR, _zfill, 0)

        def _zero(z, cc):
            cid = s + NS * z

            @pl.when(cid < n_zc)
            def _():
                pltpu.sync_copy(zb, acc_sh.at[pl.ds(cid * ZR, ZR)])

            @pl.when(cid == n_zc)
            def _():
                pltpu.sync_copy(zb.at[pl.ds(0, TAIL)],
                                acc_sh.at[pl.ds(n_zc * ZR, TAIL)])
            return cc
        lax.fori_loop(0, n_zc // NS + 1, _zero, 0)
        plsc.subcore_barrier()

        def _super(si, carry):
            base = s * ept + si * SUPER
            pltpu.sync_copy(src_hbm.at[pl.ds(base, SUPER)], srcv)
            pltpu.sync_copy(dst_hbm.at[pl.ds(base, SUPER)], dstv)
            pltpu.sync_copy(exf_hbm.at[pl.ds(k * E + base, SUPER)],
                            exv.at[pl.ds(0, SUPER)])

            pltpu.async_copy(h_hbm.at[dstv.at[pl.ds(0, C)]], rows[0], sem_g[0])

            def _pair(pr, cc):
                for b in (0, 1):
                    j = 2 * pr + b
                    o = 1 - b
                    pltpu.make_async_copy(
                        h_hbm.at[dstv.at[pl.ds(0, C)]], rows[b], sem_g[b]).wait()

                    @pl.when(j + 1 < NSUB)
                    def _():
                        pltpu.async_copy(
                            h_hbm.at[dstv.at[pl.ds((j + 1) * C, C)]],
                            rows[o], sem_g[o])

                    @pl.when(pr >= 1)
                    def _():
                        pltpu.make_async_copy(
                            sbuf[b], acc_sh.at[sidx[b]], sem_s[b]).wait()

                    @plsc.parallel_loop(0, C, unroll=4)
                    def _scale(i):
                        ev = exv[pl.ds(j * C + i, L)]
                        bc = jnp.broadcast_to(ev[0], (L,))
                        for jj in range(H // L):
                            sbuf[b][i, pl.ds(jj * L, L)] = (
                                rows[b][i, pl.ds(jj * L, L)] * bc)
                    for g in range(C // L):
                        sidx[b][pl.ds(g * L, L)] = srcv[pl.ds(j * C + g * L, L)]
                    pltpu.async_copy(sbuf[b], acc_sh.at[sidx[b]], sem_s[b],
                                     add=True)
                return cc
            lax.fori_loop(0, NSUB // 2, _pair, 0)
            pltpu.make_async_copy(sbuf[0], acc_sh.at[sidx[0]], sem_s[0]).wait()
            pltpu.make_async_copy(sbuf[1], acc_sh.at[sidx[1]], sem_s[1]).wait()
            return carry
        lax.fori_loop(0, n_super, _super, 0)
        plsc.subcore_barrier()

        def _wchunk(r0, nr, dlen):
            pltpu.sync_copy(acc_sh.at[pl.ds(r0, nr)], zb.at[pl.ds(0, nr)])
            pltpu.sync_copy(denomp_hbm.at[pl.ds(r0 * 4, dlen)],
                            dm0.at[pl.ds(0, dlen)])
            pltpu.sync_copy(denomp_hbm.at[pl.ds(N4 + r0 * 4, dlen)],
                            dm1.at[pl.ds(0, dlen)])
            for g in range(dlen // L):
                dm0[pl.ds(g * L, L)] = (dm0[pl.ds(g * L, L)]
                                        + dm1[pl.ds(g * L, L)])
            for g in range(nr // L):
                d16 = plsc.load_gather(dm0, [(g * L + lanes) * 4 + k])
                r16 = jnp.where(d16 > 0.0, 1.0 / d16, 0.0)
                for l in range(L):
                    bc = jnp.broadcast_to(r16[l], (L,))
                    for jj in range(H // L):
                        zb[g * L + l, pl.ds(jj * L, L)] = (
                            zb[g * L + l, pl.ds(jj * L, L)] * bc)
            pltpu.sync_copy(zb.at[pl.ds(0, nr)], agg_hbm.at[k, pl.ds(r0, nr)])

        def _wout(z, cc):
            cid = s + NS * z

            @pl.when(cid < n_zc)
            def _():
                _wchunk(cid * ZR, ZR, ZR * 4)

            @pl.when(cid == n_zc)
            def _():
                _wchunk(n_zc * ZR, TAIL, TAIL * 4)
            return cc
        lax.fori_loop(0, n_zc // NS + 1, _wout, 0)
        plsc.subcore_barrier()


def _out_body(h_ref, agg_ref, o_ref):
    parts = [h_ref[...] + agg_ref[kk] for kk in range(agg_ref.shape[0])]
    o_ref[...] = jnp.concatenate(parts, axis=1)


def kernel(feats, adjacency_matrix, adj_ind, W, W_b, attn_w, attn_b):
    N, F = feats.shape
    H = W.shape[0]
    HEADS = attn_w.shape[0]
    E = adj_ind.shape[1]
    del adjacency_matrix

    src = adj_ind[0].astype(jnp.int32)
    dst = adj_ind[1].astype(jnp.int32)

    Wt = W.T
    wb2 = W_b.reshape(1, H)
    P = jnp.concatenate([attn_w[:, :H].T, attn_w[:, H:].T], axis=1)
    pb2 = jnp.concatenate([attn_b, jnp.zeros((HEADS,), jnp.float32)]).reshape(1, 2 * HEADS)

    R = 1000
    h, a8 = pl.pallas_call(
        _dense_body,
        grid=(N // R,),
        in_specs=[
            pl.BlockSpec((R, F), lambda i: (i, 0)),
            pl.BlockSpec((F, H), lambda i: (0, 0)),
            pl.BlockSpec((1, H), lambda i: (0, 0)),
            pl.BlockSpec((H, 2 * HEADS), lambda i: (0, 0)),
            pl.BlockSpec((1, 2 * HEADS), lambda i: (0, 0)),
        ],
        out_specs=[
            pl.BlockSpec((R, H), lambda i: (i, 0)),
            pl.BlockSpec((R, 2 * HEADS), lambda i: (i, 0)),
        ],
        out_shape=[
            jax.ShapeDtypeStruct((N, H), jnp.float32),
            jax.ShapeDtypeStruct((N, 2 * HEADS), jnp.float32),
        ],
    )(feats, Wt, wb2, P, pb2)

    a8f = a8.reshape(-1)

    mesh = plsc.VectorSubcoreMesh(core_axis_name="c", subcore_axis_name="s",
                                  num_cores=NC, num_subcores=NS)
    sc_params = pltpu.CompilerParams(needs_layout_passes=False)

    edge_k = pl.kernel(
        functools.partial(_edge_body, E),
        out_type=(
            jax.ShapeDtypeStruct((HEADS * E,), jnp.float32),
            jax.ShapeDtypeStruct((NC * N * HEADS,), jnp.float32),
        ),
        mesh=mesh,
        scratch_types=(
            pltpu.VMEM((N * 2 * HEADS,), jnp.float32),
            pltpu.VMEM((80,), jnp.int32),
            pltpu.VMEM((80,), jnp.int32),
            pltpu.VMEM((HEADS, 80), jnp.float32),
            pltpu.VMEM((80,), jnp.int32),
            pltpu.VMEM((N * HEADS // 10,), jnp.float32),
            pltpu.VMEM_SHARED((N * HEADS,), jnp.float32),
        ),
        compiler_params=sc_params,
    )
    exf, denomp = edge_k(src, dst, a8f)

    agg_k = pl.kernel(
        functools.partial(_agg_body, N, E, H),
        out_type=jax.ShapeDtypeStruct((HEADS, N, H), jnp.float32),
        mesh=mesh,
        scratch_types=(
            pltpu.VMEM((800,), jnp.int32),
            pltpu.VMEM((800,), jnp.int32),
            pltpu.VMEM((816,), jnp.float32),
            pltpu.VMEM((80,), jnp.int32),
            pltpu.VMEM((80,), jnp.int32),
            pltpu.VMEM((80, H), jnp.float32),
            pltpu.VMEM((80, H), jnp.float32),
            pltpu.VMEM((80, H), jnp.float32),
            pltpu.VMEM((80, H), jnp.float32),
            pltpu.VMEM((32, H), jnp.float32),
            pltpu.VMEM((128,), jnp.float32),
            pltpu.VMEM((128,), jnp.float32),
            pltpu.VMEM_SHARED((N, H), jnp.float32),
            pltpu.SemaphoreType.DMA,
            pltpu.SemaphoreType.DMA,
            pltpu.SemaphoreType.DMA,
            pltpu.SemaphoreType.DMA,
        ),
        compiler_params=sc_params,
    )
    agg = agg_k(src, dst, exf, denomp, h)

    out = pl.pallas_call(
        _out_body,
        grid=(N // R,),
        in_specs=[
            pl.BlockSpec((R, H), lambda i: (i, 0)),
            pl.BlockSpec((HEADS, R, H), lambda i: (0, i, 0)),
        ],
        out_specs=pl.BlockSpec((R, HEADS * H), lambda i: (i, 0)),
        out_shape=jax.ShapeDtypeStruct((N, HEADS * H), jnp.float32),
    )(h, agg)
    return out
